# baseline (speedup 1.0000x reference)
import jax
import jax.numpy as jnp
from jax import lax
from jax.experimental import pallas as pl
from jax.experimental.pallas import tpu as pltpu
from jax.experimental.pallas import tpu_sc as plsc

E = 320_000
V = 10_000
D = 128
W = D // 2
NC = 2
NS = 16
NW = NC * NS
EW = E // NW
C = 80
NCHUNK = EW // C
NB = 2
L = 16


def _sc_body(z_hbm, ei_hbm, out_hbm,
             sidx_v, didx_v, srows, drows, ovals, gsem):
    wid = lax.axis_index("s") * NC + lax.axis_index("c")
    base = pl.multiple_of(wid * EW, 8)

    pltpu.sync_copy(ei_hbm.at[0, pl.ds(base, EW)], sidx_v)
    pltpu.sync_copy(ei_hbm.at[1, pl.ds(base, EW)], didx_v)

    def issue_gather(c, slot):
        off = pl.multiple_of(c * C, 8)
        pltpu.async_copy(z_hbm.at[sidx_v.at[pl.ds(off, C)]],
                         srows.at[slot], gsem.at[slot])
        pltpu.async_copy(z_hbm.at[didx_v.at[pl.ds(off, C)]],
                         drows.at[slot], gsem.at[slot])

    def wait_gather(c, slot):
        off = pl.multiple_of(c * C, 8)
        pltpu.make_async_copy(z_hbm.at[sidx_v.at[pl.ds(off, C)]],
                              srows.at[slot], gsem.at[slot]).wait()
        pltpu.make_async_copy(z_hbm.at[didx_v.at[pl.ds(off, C)]],
                              drows.at[slot], gsem.at[slot]).wait()

    lanes = lax.iota(jnp.int32, L)

    def edge_dot(rows_s, rows_d, e):
        ps = []
        for k in range(W // L):
            s = plsc.bitcast(rows_s[e, pl.ds(k * L, L)], jnp.bfloat16)
            d = plsc.bitcast(rows_d[e, pl.ds(k * L, L)], jnp.bfloat16)
            ps.append(s * d)
        c0 = ps[0] + ps[1]
        c1 = ps[2] + ps[3]
        p0, p1 = plsc.unpack(c0 + c1,
                             format=plsc.PackFormat.INTERLEAVED,
                             preferred_element_type=jnp.float32)
        return jnp.sum(p0 + p1)

    def compute(c, slot):
        rows_s = srows.at[slot]
        rows_d = drows.at[slot]
        obase = c * C

        def group(g, _):
            parts = [jnp.where(lanes == e16,
                               edge_dot(rows_s, rows_d, g * L + e16), 0.0)
                     for e16 in range(L)]
            while len(parts) > 1:
                parts = [parts[i] + parts[i + 1]
                         for i in range(0, len(parts), 2)]
            res = parts[0]
            off = pl.multiple_of(obase, 8) + g * L
            ovals[pl.ds(off, L)] = 1.0 / (1.0 + jnp.exp(-res))
            return 0

        lax.fori_loop(0, C // L, group, 0)

    issue_gather(0, 0)

    def body(j, _):
        a = 2 * j
        b = a + 1
        issue_gather(b, 1)
        wait_gather(a, 0)
        compute(a, 0)
        issue_gather(b + 1, 0)
        wait_gather(b, 1)
        compute(b, 1)
        return 0

    lax.fori_loop(0, (NCHUNK - 1) // 2, body, 0)
    wait_gather(NCHUNK - 1, 0)
    compute(NCHUNK - 1, 0)

    pltpu.sync_copy(ovals, out_hbm.at[pl.ds(base, EW)])


@jax.jit
def _decode(zpacked, edge_index):
    mesh = plsc.VectorSubcoreMesh(
        core_axis_name="c", subcore_axis_name="s",
        num_cores=NC, num_subcores=NS,
    )
    return pl.kernel(
        _sc_body,
        out_type=jax.ShapeDtypeStruct((E,), jnp.float32),
        mesh=mesh,
        scratch_types=[
            pltpu.VMEM((EW,), jnp.int32),
            pltpu.VMEM((EW,), jnp.int32),
            pltpu.VMEM((NB, C, W), jnp.int32),
            pltpu.VMEM((NB, C, W), jnp.int32),
            pltpu.VMEM((EW,), jnp.float32),
            pltpu.SemaphoreType.DMA((NB,)),
        ],
        compiler_params=pltpu.CompilerParams(needs_layout_passes=False,
                                             use_tc_tiling_on_sc=False),
    )(zpacked, edge_index)


def kernel(z, edge_index):
    u = lax.bitcast_convert_type(z, jnp.uint32)
    bits = (u + 0x7FFF + ((u >> 16) & 1)) >> 16
    packed = lax.bitcast_convert_type(bits[:, :W] | (bits[:, W:] << 16),
                                      jnp.int32)
    return _decode(packed, edge_index.astype(jnp.int32))


# speedup vs baseline: 1.0107x; 1.0107x over previous
import jax
import jax.numpy as jnp
from jax import lax
from jax.experimental import pallas as pl
from jax.experimental.pallas import tpu as pltpu
from jax.experimental.pallas import tpu_sc as plsc

E = 320_000
V = 10_000
D = 128
W = D // 2
NC = 2
NS = 16
NW = NC * NS
EW = E // NW
C = 80
NCHUNK = EW // C
NB = 2
L = 16


def _sc_body(z_hbm, ei_hbm, out_hbm,
             sidx_v, didx_v, srows, drows, ovals, gsem):
    wid = lax.axis_index("s") * NC + lax.axis_index("c")
    base = pl.multiple_of(wid * EW, 8)

    pltpu.sync_copy(ei_hbm.at[0, pl.ds(base, EW)], sidx_v)
    pltpu.sync_copy(ei_hbm.at[1, pl.ds(base, EW)], didx_v)

    def issue_gather(c, slot):
        off = pl.multiple_of(c * C, 8)
        pltpu.async_copy(z_hbm.at[sidx_v.at[pl.ds(off, C)]],
                         srows.at[slot], gsem.at[slot])
        pltpu.async_copy(z_hbm.at[didx_v.at[pl.ds(off, C)]],
                         drows.at[slot], gsem.at[slot])

    def wait_gather(c, slot):
        off = pl.multiple_of(c * C, 8)
        pltpu.make_async_copy(z_hbm.at[sidx_v.at[pl.ds(off, C)]],
                              srows.at[slot], gsem.at[slot]).wait()
        pltpu.make_async_copy(z_hbm.at[didx_v.at[pl.ds(off, C)]],
                              drows.at[slot], gsem.at[slot]).wait()

    lanes = lax.iota(jnp.int32, L)

    def edge_dot(rows_s, rows_d, e):
        ps = []
        for k in range(W // L):
            s = plsc.bitcast(rows_s[e, pl.ds(k * L, L)], jnp.bfloat16)
            d = plsc.bitcast(rows_d[e, pl.ds(k * L, L)], jnp.bfloat16)
            ps.append(s * d)
        c0 = ps[0] + ps[1]
        c1 = ps[2] + ps[3]
        p0, p1 = plsc.unpack(c0 + c1,
                             format=plsc.PackFormat.INTERLEAVED,
                             preferred_element_type=jnp.float32)
        return jnp.sum(p0 + p1)

    def compute(c, slot):
        rows_s = srows.at[slot]
        rows_d = drows.at[slot]
        obase = c * C

        def group(g, _):
            res = jnp.zeros((L,), jnp.float32)
            for e16 in range(L):
                e = g * L + e16
                res = jnp.where(lanes == e16, edge_dot(rows_s, rows_d, e),
                                res)
            off = pl.multiple_of(obase, 8) + g * L
            ovals[pl.ds(off, L)] = 1.0 / (1.0 + jnp.exp(-res))
            return 0

        lax.fori_loop(0, C // L, group, 0)

    issue_gather(0, 0)

    def body(j, _):
        a = 2 * j
        b = a + 1
        issue_gather(b, 1)
        wait_gather(a, 0)
        compute(a, 0)
        issue_gather(b + 1, 0)
        wait_gather(b, 1)
        compute(b, 1)
        return 0

    lax.fori_loop(0, (NCHUNK - 1) // 2, body, 0)
    wait_gather(NCHUNK - 1, 0)
    compute(NCHUNK - 1, 0)

    pltpu.sync_copy(ovals, out_hbm.at[pl.ds(base, EW)])


@jax.jit
def _decode(zpacked, edge_index):
    mesh = plsc.VectorSubcoreMesh(
        core_axis_name="c", subcore_axis_name="s",
        num_cores=NC, num_subcores=NS,
    )
    return pl.kernel(
        _sc_body,
        out_type=jax.ShapeDtypeStruct((E,), jnp.float32),
        mesh=mesh,
        scratch_types=[
            pltpu.VMEM((EW,), jnp.int32),
            pltpu.VMEM((EW,), jnp.int32),
            pltpu.VMEM((NB, C, W), jnp.int32),
            pltpu.VMEM((NB, C, W), jnp.int32),
            pltpu.VMEM((EW,), jnp.float32),
            pltpu.SemaphoreType.DMA((NB,)),
        ],
        compiler_params=pltpu.CompilerParams(needs_layout_passes=False,
                                             use_tc_tiling_on_sc=False),
    )(zpacked, edge_index)


def kernel(z, edge_index):
    u = lax.bitcast_convert_type(z, jnp.uint32)
    bits = (u + 0x7FFF + ((u >> 16) & 1)) >> 16
    packed = lax.bitcast_convert_type(bits[:, :W] | (bits[:, W:] << 16),
                                      jnp.int32)
    return _decode(packed, edge_index.astype(jnp.int32))
